# Initial kernel scaffold; baseline (speedup 1.0000x reference)
import jax
import jax.numpy as jnp
from jax.experimental import pallas as pl


def kernel(x, codebook):
    raise NotImplementedError("write your pallas kernel here")


# speedup vs baseline: 1.5276x; 1.5276x over previous
import functools

import jax
import jax.numpy as jnp
from jax import lax
from jax.experimental import pallas as pl
from jax.experimental.pallas import tpu as pltpu
from jax.experimental.pallas import tpu_sc as plsc

EPS = 1e-12
BETA = 0.25
TM = 4096
TK = 4096
NCH = 8


def _norm_rows_kernel(cb_ref, out_ref):
    cb = cb_ref[...]
    nrm = jnp.sqrt(jnp.sum(cb * cb, axis=-1, keepdims=True)) + EPS
    out_ref[...] = cb / nrm


def _argmax_loss_kernel(z_ref, cb_ref, idx_ref, loss_ref, en_ref,
                        zn_scr, ss_scr, rmax_scr, ridx_scr):
    m = pl.program_id(0)
    k = pl.program_id(1)
    nm = pl.num_programs(0)
    nk = pl.num_programs(1)
    n_tot = nm * TM
    c_dim = z_ref.shape[1]

    @pl.when(jnp.logical_and(m == 0, k == 0))
    def _():
        loss_ref[...] = jnp.zeros_like(loss_ref)

    @pl.when(k == 0)
    def _():
        z = z_ref[...]
        ss = jnp.sum(z * z, axis=-1, keepdims=True)
        ss_scr[...] = ss
        zn_scr[...] = z / (jnp.sqrt(ss) + EPS)
        rmax_scr[...] = jnp.full((TM, 1), -jnp.inf, dtype=jnp.float32)
        ridx_scr[...] = jnp.zeros((TM, 1), dtype=jnp.int32)

    cb = cb_ref[...]
    en = cb / (jnp.sqrt(jnp.sum(cb * cb, axis=-1, keepdims=True)) + EPS)
    en_ref[...] = en

    zn = zn_scr[...]
    ck = TK // NCH
    iota_f = lax.broadcasted_iota(jnp.int32, (TM, ck), 1).astype(jnp.float32)
    for j in range(NCH):
        e = en[j * ck:(j + 1) * ck, :]
        sim = lax.dot_general(zn, e, (((1,), (1,)), ((), ())),
                              preferred_element_type=jnp.float32)
        lmax = jnp.max(sim, axis=1, keepdims=True)
        loc = jnp.min(jnp.where(sim == lmax, iota_f, jnp.float32(3.0e38)),
                      axis=1, keepdims=True)
        lidx = loc.astype(jnp.int32) + (k * TK + j * ck)
        better = lmax > rmax_scr[...]
        ridx_scr[...] = jnp.where(better, lidx, ridx_scr[...])
        rmax_scr[...] = jnp.where(better, lmax, rmax_scr[...])

    @pl.when(k == nk - 1)
    def _():
        idx_ref[...] = ridx_scr[...]
        ss = ss_scr[...]
        row_loss = 1.0 + ss - 2.0 * rmax_scr[...] * (jnp.sqrt(ss) + EPS)
        loss_ref[...] = loss_ref[...] + jnp.sum(row_loss).reshape(1, 1)

    @pl.when(jnp.logical_and(m == nm - 1, k == nk - 1))
    def _():
        scale = (1.0 + BETA) / (n_tot * c_dim)
        loss_ref[...] = loss_ref[...] * scale


@functools.lru_cache(maxsize=None)
def _make_sc_gather(v_rows, d_cols, n_rows):
    info = plsc.get_sparse_core_info()
    nw = info.num_cores * info.num_subcores
    bpw = n_rows // nw
    mesh = plsc.VectorSubcoreMesh(core_axis_name="c", subcore_axis_name="s")

    nchg = 4
    cs = bpw // nchg

    @functools.partial(
        pl.kernel, mesh=mesh,
        out_type=jax.ShapeDtypeStruct((n_rows, d_cols), jnp.float32),
        scratch_types=(
            [pltpu.VMEM((bpw,), jnp.int32)]
            + [pltpu.VMEM((cs, d_cols), jnp.float32) for _ in range(nchg)]
            + [pltpu.SemaphoreType.DMA for _ in range(2 * nchg)]
        ),
    )
    def gather_k(en_hbm, idx_hbm, out_hbm, idx_v, *bufs_and_sems):
        bufs = bufs_and_sems[:nchg]
        gsems = bufs_and_sems[nchg:2 * nchg]
        ssems = bufs_and_sems[2 * nchg:]
        wid = lax.axis_index("s") * info.num_cores + lax.axis_index("c")
        base = wid * bpw
        pltpu.sync_copy(idx_hbm.at[pl.ds(base, bpw)], idx_v)
        gh = [pltpu.async_copy(en_hbm.at[idx_v.at[pl.ds(ch * cs, cs)]],
                               bufs[ch], gsems[ch])
              for ch in range(nchg)]
        sh = []
        for ch in range(nchg):
            gh[ch].wait()
            sh.append(pltpu.async_copy(
                bufs[ch], out_hbm.at[pl.ds(base + ch * cs, cs)], ssems[ch]))
        for h in sh:
            h.wait()

    return gather_k


def kernel(x, codebook):
    b, c, l = x.shape
    kk, _ = codebook.shape
    n = b * l
    z = jnp.transpose(x, (0, 2, 1)).reshape(n, c)

    idx2, loss, e_n = pl.pallas_call(
        _argmax_loss_kernel,
        grid=(n // TM, kk // TK),
        in_specs=[
            pl.BlockSpec((TM, c), lambda m, k: (m, 0)),
            pl.BlockSpec((TK, c), lambda m, k: (k, 0)),
        ],
        out_specs=[
            pl.BlockSpec((TM, 1), lambda m, k: (m, 0)),
            pl.BlockSpec((1, 1), lambda m, k: (0, 0)),
            pl.BlockSpec((TK, c), lambda m, k: (k, 0)),
        ],
        out_shape=[
            jax.ShapeDtypeStruct((n, 1), jnp.int32),
            jax.ShapeDtypeStruct((1, 1), jnp.float32),
            jax.ShapeDtypeStruct((kk, c), jnp.float32),
        ],
        scratch_shapes=[
            pltpu.VMEM((TM, c), jnp.float32),
            pltpu.VMEM((TM, 1), jnp.float32),
            pltpu.VMEM((TM, 1), jnp.float32),
            pltpu.VMEM((TM, 1), jnp.int32),
        ],
        compiler_params=pltpu.CompilerParams(
            dimension_semantics=("arbitrary", "arbitrary")),
    )(z, codebook)

    idx = idx2.reshape(n)
    q = _make_sc_gather(kk, c, n)(e_n, idx)
    quantized = jnp.transpose(q.reshape(b, l, c), (0, 2, 1))
    return quantized, loss[0, 0]


# speedup vs baseline: 1.5990x; 1.0467x over previous
import functools

import jax
import jax.numpy as jnp
from jax import lax
from jax.experimental import pallas as pl
from jax.experimental.pallas import tpu as pltpu
from jax.experimental.pallas import tpu_sc as plsc

EPS = 1e-12
BETA = 0.25
TM = 4096
TK = 4096
NCH = 8


def _norm_rows_kernel(cb_ref, out_ref):
    cb = cb_ref[...]
    nrm = jnp.sqrt(jnp.sum(cb * cb, axis=-1, keepdims=True)) + EPS
    out_ref[...] = cb / nrm


def _argmax_loss_kernel(z_ref, cb_ref, idx_ref, loss_ref, en_ref,
                        zn_scr, ss_scr, rmax_scr, ridx_scr):
    m = pl.program_id(0)
    k = pl.program_id(1)
    nm = pl.num_programs(0)
    nk = pl.num_programs(1)
    n_tot = nm * TM
    c_dim = z_ref.shape[1]

    @pl.when(jnp.logical_and(m == 0, k == 0))
    def _():
        loss_ref[...] = jnp.zeros_like(loss_ref)

    @pl.when(k == 0)
    def _():
        z = z_ref[...]
        ss = jnp.sum(z * z, axis=-1, keepdims=True)
        ss_scr[...] = ss
        zn_scr[...] = z / (jnp.sqrt(ss) + EPS)
        rmax_scr[...] = jnp.full((TM, 1), -jnp.inf, dtype=jnp.float32)
        ridx_scr[...] = jnp.zeros((TM, 1), dtype=jnp.int32)

    cb = cb_ref[...]
    en = cb / (jnp.sqrt(jnp.sum(cb * cb, axis=-1, keepdims=True)) + EPS)
    en_ref[...] = en

    zn = zn_scr[...]
    ck = TK // NCH
    iota_f = lax.broadcasted_iota(jnp.int32, (TM, ck), 1).astype(jnp.float32)
    for j in range(NCH):
        e = en[j * ck:(j + 1) * ck, :]
        sim = lax.dot_general(zn, e, (((1,), (1,)), ((), ())),
                              preferred_element_type=jnp.float32)
        lmax = jnp.max(sim, axis=1, keepdims=True)
        loc = jnp.min(jnp.where(sim == lmax, iota_f, jnp.float32(3.0e38)),
                      axis=1, keepdims=True)
        lidx = loc.astype(jnp.int32) + (k * TK + j * ck)
        better = lmax > rmax_scr[...]
        ridx_scr[...] = jnp.where(better, lidx, ridx_scr[...])
        rmax_scr[...] = jnp.where(better, lmax, rmax_scr[...])

    @pl.when(k == nk - 1)
    def _():
        idx_ref[...] = ridx_scr[...].reshape(idx_ref.shape)
        ss = ss_scr[...]
        row_loss = 1.0 + ss - 2.0 * rmax_scr[...] * (jnp.sqrt(ss) + EPS)
        loss_ref[...] = loss_ref[...] + jnp.sum(row_loss).reshape(1, 1)

    @pl.when(jnp.logical_and(m == nm - 1, k == nk - 1))
    def _():
        scale = (1.0 + BETA) / (n_tot * c_dim)
        loss_ref[...] = loss_ref[...] * scale


@functools.lru_cache(maxsize=None)
def _make_sc_gather(v_rows, d_cols, n_rows):
    info = plsc.get_sparse_core_info()
    nw = info.num_cores * info.num_subcores
    bpw = n_rows // nw
    mesh = plsc.VectorSubcoreMesh(core_axis_name="c", subcore_axis_name="s")

    nchg = 4
    cs = bpw // nchg

    @functools.partial(
        pl.kernel, mesh=mesh,
        out_type=jax.ShapeDtypeStruct((n_rows, d_cols), jnp.float32),
        scratch_types=(
            [pltpu.VMEM((bpw,), jnp.int32)]
            + [pltpu.VMEM((cs, d_cols), jnp.float32) for _ in range(nchg)]
            + [pltpu.SemaphoreType.DMA for _ in range(2 * nchg)]
        ),
    )
    def gather_k(en_hbm, idx_hbm, out_hbm, idx_v, *bufs_and_sems):
        bufs = bufs_and_sems[:nchg]
        gsems = bufs_and_sems[nchg:2 * nchg]
        ssems = bufs_and_sems[2 * nchg:]
        wid = lax.axis_index("s") * info.num_cores + lax.axis_index("c")
        base = wid * bpw
        pltpu.sync_copy(idx_hbm.at[pl.ds(base, bpw)], idx_v)
        gh = [pltpu.async_copy(en_hbm.at[idx_v.at[pl.ds(ch * cs, cs)]],
                               bufs[ch], gsems[ch])
              for ch in range(nchg)]
        sh = []
        for ch in range(nchg):
            gh[ch].wait()
            sh.append(pltpu.async_copy(
                bufs[ch], out_hbm.at[pl.ds(base + ch * cs, cs)], ssems[ch]))
        for h in sh:
            h.wait()

    return gather_k


def kernel(x, codebook):
    b, c, l = x.shape
    kk, _ = codebook.shape
    n = b * l
    z = jnp.transpose(x, (0, 2, 1)).reshape(n, c)

    idx2, loss, e_n = pl.pallas_call(
        _argmax_loss_kernel,
        grid=(n // TM, kk // TK),
        in_specs=[
            pl.BlockSpec((TM, c), lambda m, k: (m, 0)),
            pl.BlockSpec((TK, c), lambda m, k: (k, 0)),
        ],
        out_specs=[
            pl.BlockSpec((TM // 128, 128), lambda m, k: (m, 0)),
            pl.BlockSpec((1, 1), lambda m, k: (0, 0)),
            pl.BlockSpec((TK, c), lambda m, k: (k, 0)),
        ],
        out_shape=[
            jax.ShapeDtypeStruct((n // 128, 128), jnp.int32),
            jax.ShapeDtypeStruct((1, 1), jnp.float32),
            jax.ShapeDtypeStruct((kk, c), jnp.float32),
        ],
        scratch_shapes=[
            pltpu.VMEM((TM, c), jnp.float32),
            pltpu.VMEM((TM, 1), jnp.float32),
            pltpu.VMEM((TM, 1), jnp.float32),
            pltpu.VMEM((TM, 1), jnp.int32),
        ],
        compiler_params=pltpu.CompilerParams(
            dimension_semantics=("arbitrary", "arbitrary")),
    )(z, codebook)

    idx = idx2.reshape(n)
    q = _make_sc_gather(kk, c, n)(e_n, idx)
    quantized = jnp.transpose(q.reshape(b, l, c), (0, 2, 1))
    return quantized, loss[0, 0]
